# speedup vs baseline: 32.8629x; 32.8629x over previous
import math

import jax
import jax.numpy as jnp
from jax.experimental import pallas as pl

B = 2
T = 2048
D_MODEL = 1024
SKILL_DIM = 128
MAX_SKILLS = 4096
TOP_K = 8


def _fused_kernel(h_last_ref, embeds_ref, wq_ref, bq_ref, wk_ref, bk_ref,
                  wkv_ref, bkv_ref, wo_ref, h_exec_ref, skill_ref):
    q = h_last_ref[...] @ wq_ref[...] + bq_ref[...]
    keys = embeds_ref[...] @ wk_ref[...] + bk_ref[...]
    s = (q @ keys.T) * (1.0 / math.sqrt(SKILL_DIM))

    iota = jax.lax.broadcasted_iota(jnp.int32, (B, MAX_SKILLS), 1)
    work = s
    sel = jnp.zeros((B, MAX_SKILLS), dtype=jnp.bool_)
    for _ in range(TOP_K):
        m = jnp.max(work, axis=1, keepdims=True)
        cand = jnp.where(work == m, iota, MAX_SKILLS)
        amin = jnp.min(cand, axis=1, keepdims=True)
        pick = iota == amin
        sel = jnp.logical_or(sel, pick)
        work = jnp.where(pick, -jnp.inf, work)

    logits = jnp.where(sel, s, -jnp.inf)
    mx = jnp.max(logits, axis=1, keepdims=True)
    e = jnp.exp(logits - mx)
    w = e / jnp.sum(e, axis=1, keepdims=True)
    skill = w @ embeds_ref[...]

    kv = skill @ wkv_ref[...] + bkv_ref[...]
    v = kv[:, D_MODEL:]
    row = v @ wo_ref[...]

    skill_ref[...] = skill
    h_exec_ref[...] = jnp.broadcast_to(row[:, None, :], (B, T, D_MODEL))


def kernel(h, embeds, Wq_r, bq_r, Wk_r, bk_r, Wc, bc, gate, Wkv, bkv, Wq_a, Wo):
    h_last = h[:, -1]
    h_exec, skill = pl.pallas_call(
        _fused_kernel,
        out_shape=(
            jax.ShapeDtypeStruct((B, T, D_MODEL), jnp.float32),
            jax.ShapeDtypeStruct((B, SKILL_DIM), jnp.float32),
        ),
    )(h_last, embeds, Wq_r, bq_r.reshape(1, SKILL_DIM), Wk_r,
      bk_r.reshape(1, SKILL_DIM), Wkv, bkv.reshape(1, 2 * D_MODEL), Wo)
    return (h_exec, skill)
